# Initial kernel scaffold; baseline (speedup 1.0000x reference)
import jax
import jax.numpy as jnp
from jax.experimental import pallas as pl


def kernel(x, idx, W0, b0, W1, b1, W2, b2):
    raise NotImplementedError("write your pallas kernel here")


# speedup vs baseline: 6.2092x; 6.2092x over previous
import jax
import jax.numpy as jnp
from jax.experimental import pallas as pl
from jax.experimental.pallas import tpu as pltpu
from jax.experimental.pallas import tpu_sc as plsc

_BLK = 1000
_WIN = 256


def _pre_body(xt_ref, a_ref, b_ref, gp_ref, p0_ref, c1_ref, c2_ref):
    pre = jax.lax.dot_general(
        xt_ref[...], a_ref[...], (((1,), (0,)), ((), ())),
        preferred_element_type=jnp.float32)
    pre = pre + b_ref[...]
    g = pre.shape[1] // 4
    gp_ref[...] = pre[:, :2 * g]
    p0_ref[...] = pre[:, g:2 * g]
    c1_ref[...] = pre[:, 2 * g:3 * g]
    c2_ref[...] = pre[:, 3 * g:]


def _edge_body(gat_ref, p0_ref, c1_ref, c2_ref, w1_ref, w2a_ref, w2b_ref,
               o_ref):
    blk, g = p0_ref.shape
    k = gat_ref.shape[0] // blk
    gat = gat_ref[...][:, :g].reshape(blk, k, g)
    h0 = jnp.maximum(gat + p0_ref[...][:, None, :], 0.0)
    h0f = h0.reshape(blk * k, g)
    t1 = jax.lax.dot_general(h0f, w1_ref[...], (((1,), (0,)), ((), ())),
                             preferred_element_type=jnp.float32)
    h1 = jnp.maximum(t1.reshape(blk, k, g) + c1_ref[...][:, None, :], 0.0)
    h1f = h1.reshape(blk * k, g)
    t2 = (jax.lax.dot_general(h1f, w2a_ref[...], (((1,), (0,)), ((), ())),
                              preferred_element_type=jnp.float32)
          + jax.lax.dot_general(h0f, w2b_ref[...], (((1,), (0,)), ((), ())),
                                preferred_element_type=jnp.float32))
    h2 = t2.reshape(blk, k, g) + c2_ref[...][:, None, :]
    o_ref[...] = jnp.concatenate(
        [jnp.max(h2, axis=1), jnp.max(h1, axis=1), jnp.max(h0, axis=1)],
        axis=1)


def _sc_gather(g_rows, idx_flat):
    nk = idx_flat.shape[1]
    gdim = g_rows.shape[1]

    @pl.kernel(
        out_type=jax.ShapeDtypeStruct((nk, gdim), jnp.float32),
        mesh=plsc.VectorSubcoreMesh(core_axis_name="core",
                                    subcore_axis_name="subcore"))
    def kern(x_hbm, i_hbm, o_hbm):
        def body(i_vmem, o_vmem):
            pltpu.sync_copy(x_hbm.at[i_vmem.at[0]], o_vmem)

        pltpu.emit_pipeline(
            body,
            grid=(nk // _WIN,),
            in_specs=[pl.BlockSpec((1, _WIN), lambda i: (0, i))],
            out_specs=[pl.BlockSpec((_WIN, gdim), lambda i: (i, 0))],
            core_axis_name=("core", "subcore"),
            dimension_semantics=(pltpu.PARALLEL,),
        )(i_hbm, o_hbm)

    return kern(g_rows, idx_flat)


def kernel(x, idx, W0, b0, W1, b1, W2, b2):
    B, C, N = x.shape
    k = idx.shape[-1]
    g = W0.shape[0]

    W0a, W0b = W0[:, :C], W0[:, C:]
    W1a, W1b = W1[:, :g], W1[:, g:]
    W2a, W2b, W2c = W2[:, :g], W2[:, g:2 * g], W2[:, 2 * g:]
    A = jnp.concatenate([W0b.T, (W0a - W0b).T, W1b.T, W2c.T], axis=1)
    bA = jnp.concatenate([jnp.zeros_like(b0), b0, b1, b2])[None, :]

    xt = x[0].T

    shp = jax.ShapeDtypeStruct((N, g), jnp.float32)
    GP, P0, C1, C2 = pl.pallas_call(
        _pre_body,
        out_shape=(jax.ShapeDtypeStruct((N, 2 * g), jnp.float32),
                   shp, shp, shp),
    )(xt, A, bA)

    gathered = _sc_gather(GP, idx.reshape(1, N * k))

    nblk = N // _BLK
    spec_n = pl.BlockSpec((_BLK, g), lambda i: (i, 0))
    spec_w = pl.BlockSpec((g, g), lambda i: (0, 0))
    ymax = pl.pallas_call(
        _edge_body,
        grid=(nblk,),
        in_specs=[pl.BlockSpec((_BLK * k, 2 * g), lambda i: (i, 0)),
                  spec_n, spec_n, spec_n, spec_w, spec_w, spec_w],
        out_specs=pl.BlockSpec((_BLK, 3 * g), lambda i: (i, 0)),
        out_shape=jax.ShapeDtypeStruct((N, 3 * g), jnp.float32),
    )(gathered, P0, C1, C2, W1a.T, W2a.T, W2b.T)

    y = jnp.concatenate([ymax.T[None], x], axis=1)
    return (y, idx)


# speedup vs baseline: 8.1215x; 1.3080x over previous
import jax
import jax.numpy as jnp
from jax.experimental import pallas as pl
from jax.experimental.pallas import tpu as pltpu
from jax.experimental.pallas import tpu_sc as plsc

_BLK = 1000
_WIN = 256


def _pre_body(x_ref, a_ref, b_ref, gp_ref, p0_ref, c1_ref, c2_ref):
    pre = jax.lax.dot_general(
        x_ref[...], a_ref[...], (((0,), (0,)), ((), ())),
        preferred_element_type=jnp.float32)
    pre = pre + b_ref[...]
    g = pre.shape[1] // 4
    gp_ref[...] = pre[:, :2 * g]
    p0_ref[...] = pre[:, g:2 * g]
    c1_ref[...] = pre[:, 2 * g:3 * g]
    c2_ref[...] = pre[:, 3 * g:]


def _edge_body(gat_ref, p0_ref, c1_ref, c2_ref, w1_ref, w2a_ref, w2b_ref,
               o_ref):
    blk, g = p0_ref.shape
    k = gat_ref.shape[0] // blk
    gat = gat_ref[...][:, :g].reshape(blk, k, g)
    h0 = jnp.maximum(gat + p0_ref[...][:, None, :], 0.0)
    h0f = h0.reshape(blk * k, g)
    t1 = jax.lax.dot_general(h0f, w1_ref[...], (((1,), (0,)), ((), ())),
                             preferred_element_type=jnp.float32)
    h1 = jnp.maximum(t1.reshape(blk, k, g) + c1_ref[...][:, None, :], 0.0)
    h1f = h1.reshape(blk * k, g)
    t2 = (jax.lax.dot_general(h1f, w2a_ref[...], (((1,), (0,)), ((), ())),
                              preferred_element_type=jnp.float32)
          + jax.lax.dot_general(h0f, w2b_ref[...], (((1,), (0,)), ((), ())),
                                preferred_element_type=jnp.float32))
    h2 = t2.reshape(blk, k, g) + c2_ref[...][:, None, :]
    o_ref[...] = jnp.concatenate(
        [jnp.max(h2, axis=1), jnp.max(h1, axis=1), jnp.max(h0, axis=1)],
        axis=1)


def _finalize_body(m_ref, x_ref, o_ref):
    cg = m_ref.shape[1]
    o_ref[:cg, :] = m_ref[...].T
    o_ref[cg:, :] = x_ref[...]


def _sc_gather(g_rows, idx_flat):
    nk = idx_flat.shape[1]
    gdim = g_rows.shape[1]

    @pl.kernel(
        out_type=jax.ShapeDtypeStruct((nk, gdim), jnp.float32),
        mesh=plsc.VectorSubcoreMesh(core_axis_name="core",
                                    subcore_axis_name="subcore"))
    def kern(x_hbm, i_hbm, o_hbm):
        def body(i_vmem, o_vmem):
            pltpu.sync_copy(x_hbm.at[i_vmem.at[0]], o_vmem)

        pltpu.emit_pipeline(
            body,
            grid=(nk // _WIN,),
            in_specs=[pl.BlockSpec((1, _WIN), lambda i: (0, i))],
            out_specs=[pl.BlockSpec((_WIN, gdim), lambda i: (i, 0))],
            core_axis_name=("core", "subcore"),
            dimension_semantics=(pltpu.PARALLEL,),
        )(i_hbm, o_hbm)

    return kern(g_rows, idx_flat)


def kernel(x, idx, W0, b0, W1, b1, W2, b2):
    B, C, N = x.shape
    k = idx.shape[-1]
    g = W0.shape[0]

    W0a, W0b = W0[:, :C], W0[:, C:]
    W1a, W1b = W1[:, :g], W1[:, g:]
    W2a, W2b, W2c = W2[:, :g], W2[:, g:2 * g], W2[:, 2 * g:]
    A = jnp.concatenate([W0b.T, (W0a - W0b).T, W1b.T, W2c.T], axis=1)
    bA = jnp.concatenate([jnp.zeros_like(b0), b0, b1, b2])[None, :]

    shp = jax.ShapeDtypeStruct((N, g), jnp.float32)
    GP, P0, C1, C2 = pl.pallas_call(
        _pre_body,
        out_shape=(jax.ShapeDtypeStruct((N, 2 * g), jnp.float32),
                   shp, shp, shp),
    )(x[0], A, bA)

    gathered = _sc_gather(GP, idx.reshape(1, N * k))

    nblk = N // _BLK
    spec_n = pl.BlockSpec((_BLK, g), lambda i: (i, 0))
    spec_w = pl.BlockSpec((g, g), lambda i: (0, 0))
    ymax = pl.pallas_call(
        _edge_body,
        grid=(nblk,),
        in_specs=[pl.BlockSpec((_BLK * k, 2 * g), lambda i: (i, 0)),
                  spec_n, spec_n, spec_n, spec_w, spec_w, spec_w],
        out_specs=pl.BlockSpec((_BLK, 3 * g), lambda i: (i, 0)),
        out_shape=jax.ShapeDtypeStruct((N, 3 * g), jnp.float32),
    )(gathered, P0, C1, C2, W1a.T, W2a.T, W2b.T)

    y = pl.pallas_call(
        _finalize_body,
        out_shape=jax.ShapeDtypeStruct((3 * g + C, N), jnp.float32),
    )(ymax, x[0])

    return (y[None], idx)


# speedup vs baseline: 8.3012x; 1.0221x over previous
import jax
import jax.numpy as jnp
from jax.experimental import pallas as pl
from jax.experimental.pallas import tpu as pltpu
from jax.experimental.pallas import tpu_sc as plsc

_NA = 5120
_BLKA = 1024
_BLKB = 976
_WIN = 128


def _pre_body(x_ref, a_ref, b_ref, gp_ref, p0_ref, c1_ref, c2_ref):
    pre = jax.lax.dot_general(
        x_ref[...], a_ref[...], (((0,), (0,)), ((), ())),
        preferred_element_type=jnp.float32)
    pre = pre + b_ref[...]
    g = pre.shape[1] // 4
    gp_ref[...] = pre[:, :2 * g]
    p0_ref[...] = pre[:, g:2 * g]
    c1_ref[...] = pre[:, 2 * g:3 * g]
    c2_ref[...] = pre[:, 3 * g:]


def _edge_body(gat_ref, p0_ref, c1_ref, c2_ref, w1_ref, w2a_ref, w2b_ref,
               o_ref):
    blk, g = p0_ref.shape
    k = gat_ref.shape[0] // blk
    gat = gat_ref[...][:, :g].reshape(blk, k, g)
    h0 = jnp.maximum(gat + p0_ref[...][:, None, :], 0.0)
    h0f = h0.reshape(blk * k, g)
    t1 = jax.lax.dot_general(h0f, w1_ref[...], (((1,), (0,)), ((), ())),
                             preferred_element_type=jnp.float32)
    h1 = jnp.maximum(t1.reshape(blk, k, g) + c1_ref[...][:, None, :], 0.0)
    h1f = h1.reshape(blk * k, g)
    t2 = (jax.lax.dot_general(h1f, w2a_ref[...], (((1,), (0,)), ((), ())),
                              preferred_element_type=jnp.float32)
          + jax.lax.dot_general(h0f, w2b_ref[...], (((1,), (0,)), ((), ())),
                                preferred_element_type=jnp.float32))
    h2 = t2.reshape(blk, k, g) + c2_ref[...][:, None, :]

    def kmax(h3):
        kk = h3.shape[1]
        while kk > 1:
            half = kk // 2
            h3 = jnp.maximum(h3[:, :half, :], h3[:, half:kk, :])
            kk = half
        return h3[:, 0, :]

    o_ref[...] = jnp.concatenate([kmax(h2), kmax(h1), kmax(h0)], axis=1)


def _finalize_body(ma_ref, mb_ref, x_ref, o_ref):
    cg = ma_ref.shape[1]
    na = ma_ref.shape[0]
    o_ref[:cg, :na] = ma_ref[...].T
    o_ref[:cg, na:] = mb_ref[...].T
    o_ref[cg:, :] = x_ref[...]


def _sc_gather(g_rows, idx_flat):
    nk = idx_flat.shape[1]
    gdim = g_rows.shape[1]

    @pl.kernel(
        out_type=jax.ShapeDtypeStruct((nk, gdim), g_rows.dtype),
        mesh=plsc.VectorSubcoreMesh(core_axis_name="core",
                                    subcore_axis_name="subcore"))
    def kern(x_hbm, i_hbm, o_hbm):
        def body(i_vmem, o_vmem):
            pltpu.sync_copy(x_hbm.at[i_vmem.at[0]], o_vmem)

        pltpu.emit_pipeline(
            body,
            grid=(nk // _WIN,),
            in_specs=[pl.BlockSpec((1, _WIN), lambda i: (0, i))],
            out_specs=[pl.BlockSpec((_WIN, gdim), lambda i: (i, 0))],
            core_axis_name=("core", "subcore"),
            dimension_semantics=(pltpu.PARALLEL,),
        )(i_hbm, o_hbm)

    return kern(g_rows, idx_flat)


def kernel(x, idx, W0, b0, W1, b1, W2, b2):
    B, C, N = x.shape
    k = idx.shape[-1]
    g = W0.shape[0]

    W0a, W0b = W0[:, :C], W0[:, C:]
    W1a, W1b = W1[:, :g], W1[:, g:]
    W2a, W2b, W2c = W2[:, :g], W2[:, g:2 * g], W2[:, 2 * g:]
    A = jnp.concatenate([W0b.T, (W0a - W0b).T, W1b.T, W2c.T], axis=1)
    bA = jnp.concatenate([jnp.zeros_like(b0), b0, b1, b2])[None, :]

    shp = jax.ShapeDtypeStruct((N, g), jnp.float32)
    GP, P0, C1, C2 = pl.pallas_call(
        _pre_body,
        out_shape=(jax.ShapeDtypeStruct((N, 2 * g), jnp.float32),
                   shp, shp, shp),
    )(x[0], A, bA)

    idxf = idx.reshape(1, N * k)

    def edge_call(gat, p0, c1, c2, blk):
        n = p0.shape[0]
        spec_n = pl.BlockSpec((blk, g), lambda i: (i, 0))
        spec_w = pl.BlockSpec((g, g), lambda i: (0, 0))
        return pl.pallas_call(
            _edge_body,
            grid=(n // blk,),
            in_specs=[pl.BlockSpec((blk * k, 2 * g), lambda i: (i, 0)),
                      spec_n, spec_n, spec_n, spec_w, spec_w, spec_w],
            out_specs=pl.BlockSpec((blk, 3 * g), lambda i: (i, 0)),
            out_shape=jax.ShapeDtypeStruct((n, 3 * g), jnp.float32),
        )(gat, p0, c1, c2, W1a.T, W2a.T, W2b.T)

    na = _NA if N > _NA else N
    ga = _sc_gather(GP, idxf[:, :na * k])
    gb = _sc_gather(GP, idxf[:, na * k:])
    ymax_a = edge_call(ga, P0[:na], C1[:na], C2[:na], _BLKA)
    ymax_b = edge_call(gb, P0[na:], C1[na:], C2[na:], _BLKB)

    y = pl.pallas_call(
        _finalize_body,
        out_shape=jax.ShapeDtypeStruct((3 * g + C, N), jnp.float32),
    )(ymax_a, ymax_b, x[0])

    return (y[None], idx)


# speedup vs baseline: 8.8391x; 1.0648x over previous
import jax
import jax.numpy as jnp
from jax.experimental import pallas as pl
from jax.experimental.pallas import tpu as pltpu
from jax.experimental.pallas import tpu_sc as plsc

_NA = 5120
_BLKA = 1024
_BLKB = 976
_WIN = 128


def _pre_body(x_ref, a_ref, b_ref, gp_ref, p0_ref, c12_ref):
    pre = jax.lax.dot_general(
        x_ref[...], a_ref[...], (((0,), (0,)), ((), ())),
        preferred_element_type=jnp.float32)
    pre = pre + b_ref[...]
    g = pre.shape[1] // 4
    gp_ref[...] = pre[:, :2 * g]
    p0_ref[...] = pre[:, g:2 * g]
    c12_ref[...] = pre[:, 2 * g:]


def _edge_body(gat_ref, p0_ref, c2p_ref, w1_ref, w2_ref, o_ref):
    blk, g = p0_ref.shape
    k = gat_ref.shape[0] // blk
    gat = gat_ref[...][:, :g].reshape(blk, k, g)
    h0 = jnp.maximum(gat + p0_ref[...][:, None, :], 0.0)
    h0f = h0.reshape(blk * k, g)
    u = jax.lax.dot_general(h0f, w1_ref[...], (((1,), (0,)), ((), ())),
                            preferred_element_type=jnp.float32)
    c1z = jnp.concatenate(
        [c2p_ref[...][:, :g], jnp.zeros((blk, g), jnp.float32)], axis=1)
    v = jnp.maximum(u.reshape(blk, k, 2 * g) + c1z[:, None, :], 0.0)
    j3 = jax.lax.dot_general(
        v.reshape(blk * k, 2 * g), w2_ref[...], (((1,), (0,)), ((), ())),
        preferred_element_type=jnp.float32).reshape(blk, k, 3 * g)

    kk = k
    while kk > 1:
        half = kk // 2
        j3 = jnp.maximum(j3[:, :half, :], j3[:, half:kk, :])
        kk = half
    c2z = jnp.concatenate(
        [c2p_ref[...][:, g:], jnp.zeros((blk, 2 * g), jnp.float32)], axis=1)
    o_ref[...] = j3[:, 0, :] + c2z


def _finalize_body(ma_ref, mb_ref, x_ref, o_ref):
    cg = ma_ref.shape[1]
    na = ma_ref.shape[0]
    o_ref[:cg, :na] = ma_ref[...].T
    o_ref[:cg, na:] = mb_ref[...].T
    o_ref[cg:, :] = x_ref[...]


def _sc_gather(g_rows, idx_flat):
    nk = idx_flat.shape[1]
    gdim = g_rows.shape[1]

    @pl.kernel(
        out_type=jax.ShapeDtypeStruct((nk, gdim), g_rows.dtype),
        mesh=plsc.VectorSubcoreMesh(core_axis_name="core",
                                    subcore_axis_name="subcore"))
    def kern(x_hbm, i_hbm, o_hbm):
        def body(i_vmem, o_vmem):
            pltpu.sync_copy(x_hbm.at[i_vmem.at[0]], o_vmem)

        pltpu.emit_pipeline(
            body,
            grid=(nk // _WIN,),
            in_specs=[pl.BlockSpec((1, _WIN), lambda i: (0, i))],
            out_specs=[pl.BlockSpec((_WIN, gdim), lambda i: (i, 0))],
            core_axis_name=("core", "subcore"),
            dimension_semantics=(pltpu.PARALLEL,),
        )(i_hbm, o_hbm)

    return kern(g_rows, idx_flat)


def kernel(x, idx, W0, b0, W1, b1, W2, b2):
    B, C, N = x.shape
    k = idx.shape[-1]
    g = W0.shape[0]

    W0a, W0b = W0[:, :C], W0[:, C:]
    W1a, W1b = W1[:, :g], W1[:, g:]
    W2a, W2b, W2c = W2[:, :g], W2[:, g:2 * g], W2[:, 2 * g:]
    A = jnp.concatenate([W0b.T, (W0a - W0b).T, W1b.T, W2c.T], axis=1)
    bA = jnp.concatenate([jnp.zeros_like(b0), b0, b1, b2])[None, :]

    GP, P0, C12 = pl.pallas_call(
        _pre_body,
        out_shape=(jax.ShapeDtypeStruct((N, 2 * g), jnp.float32),
                   jax.ShapeDtypeStruct((N, g), jnp.float32),
                   jax.ShapeDtypeStruct((N, 2 * g), jnp.float32)),
    )(x[0], A, bA)

    idxf = idx.reshape(1, N * k)

    eye = jnp.eye(g, dtype=jnp.float32)
    zer = jnp.zeros((g, g), jnp.float32)
    W1p = jnp.concatenate([W1a.T, eye], axis=1)
    W2p = jnp.concatenate(
        [jnp.concatenate([W2a.T, eye, zer], axis=1),
         jnp.concatenate([W2b.T, zer, eye], axis=1)], axis=0)

    def edge_call(gat, p0, c12, blk):
        n = p0.shape[0]
        return pl.pallas_call(
            _edge_body,
            grid=(n // blk,),
            in_specs=[pl.BlockSpec((blk * k, 2 * g), lambda i: (i, 0)),
                      pl.BlockSpec((blk, g), lambda i: (i, 0)),
                      pl.BlockSpec((blk, 2 * g), lambda i: (i, 0)),
                      pl.BlockSpec((g, 2 * g), lambda i: (0, 0)),
                      pl.BlockSpec((2 * g, 3 * g), lambda i: (0, 0))],
            out_specs=pl.BlockSpec((blk, 3 * g), lambda i: (i, 0)),
            out_shape=jax.ShapeDtypeStruct((n, 3 * g), jnp.float32),
        )(gat, p0, c12, W1p, W2p)

    na = _NA if N > _NA else N
    ga = _sc_gather(GP, idxf[:, :na * k])
    gb = _sc_gather(GP, idxf[:, na * k:])
    ymax_a = edge_call(ga, P0[:na], C12[:na], _BLKA)
    ymax_b = edge_call(gb, P0[na:], C12[na:], _BLKB)

    y = pl.pallas_call(
        _finalize_body,
        out_shape=jax.ShapeDtypeStruct((3 * g + C, N), jnp.float32),
    )(ymax_a, ymax_b, x[0])

    return (y[None], idx)
